# baseline (speedup 1.0000x reference)
import functools

import jax
import jax.numpy as jnp
from jax import lax
from jax.experimental import pallas as pl
from jax.experimental.pallas import tpu as pltpu
from jax.experimental.pallas import tpu_sc as plsc

N_TOTAL = 4_000_000
N_BINS_K = 15
CHUNK = 10_000
N_CHUNKS = N_TOTAL // CHUNK
N_WORKERS = 32
VREGS_PER_CHUNK = CHUNK // 16
UNROLL = 5
TBL = 768

_mesh = plsc.VectorSubcoreMesh(core_axis_name="c", subcore_axis_name="s")


@functools.partial(
    pl.kernel,
    out_type=jax.ShapeDtypeStruct((N_WORKERS * TBL,), jnp.float32),
    mesh=_mesh,
    compiler_params=pltpu.CompilerParams(needs_layout_passes=False),
    scratch_types=[
        pltpu.VMEM((CHUNK,), jnp.int32),
        pltpu.VMEM((CHUNK,), jnp.int32),
        pltpu.VMEM((CHUNK,), jnp.float32),
        pltpu.VMEM((TBL,), jnp.float32),
    ],
)
def _ece_hist(preds_hbm, targets_hbm, confs_hbm, out_hbm,
              p_v, t_v, c_v, tbl_v):
    wid = lax.axis_index("s") * 2 + lax.axis_index("c")

    zero = jnp.zeros((16,), jnp.float32)
    for k in range(TBL // 16):
        tbl_v[pl.ds(k * 16, 16)] = zero

    lane = lax.iota(jnp.int32, 16)
    one = jnp.ones((16,), jnp.float32)
    n_mine = jnp.where(wid < (N_CHUNKS % N_WORKERS),
                       N_CHUNKS // N_WORKERS + 1,
                       N_CHUNKS // N_WORKERS)

    def chunk_body(i, carry):
        off = (wid + i * N_WORKERS) * CHUNK
        pltpu.sync_copy(preds_hbm.at[pl.ds(off, CHUNK)], p_v)
        pltpu.sync_copy(targets_hbm.at[pl.ds(off, CHUNK)], t_v)
        pltpu.sync_copy(confs_hbm.at[pl.ds(off, CHUNK)], c_v)

        def vbody(j, c2):
            for u in range(UNROLL):
                b0 = (j * UNROLL + u) * 16
                conf = c_v[pl.ds(b0, 16)]
                p = p_v[pl.ds(b0, 16)]
                t = t_v[pl.ds(b0, 16)]
                ji = (conf * jnp.float32(N_BINS_K)).astype(jnp.int32)
                ji = jnp.minimum(jnp.maximum(ji, 0), N_BINS_K - 1)
                jf = ji.astype(jnp.float32)
                blo = jf * jnp.float32(1.0 / N_BINS_K)
                bhi = (jf + 1.0) * jnp.float32(1.0 / N_BINS_K)
                binx = (ji
                        - jnp.where(conf <= blo, 1, 0)
                        + jnp.where(conf > bhi, 1, 0))
                valid = jnp.logical_and(binx >= 0, binx <= N_BINS_K - 1)
                sidx = binx * 16 + lane
                acc = jnp.where(p == t, jnp.float32(1.0), jnp.float32(0.0))
                plsc.addupdate_scatter(tbl_v, [sidx], one, mask=valid)
                plsc.addupdate_scatter(tbl_v, [sidx + 256], acc, mask=valid)
                plsc.addupdate_scatter(tbl_v, [sidx + 512], conf, mask=valid)
            return c2

        lax.fori_loop(0, VREGS_PER_CHUNK // UNROLL, vbody, 0)
        return carry

    lax.fori_loop(0, n_mine, chunk_body, 0)
    pltpu.sync_copy(tbl_v, out_hbm.at[pl.ds(wid * TBL, TBL)])


def kernel(preds, targets, confs):
    raw = _ece_hist(preds.astype(jnp.int32), targets.astype(jnp.int32),
                    confs)
    tbl = raw.reshape(N_WORKERS, 3, 256)[:, :, :N_BINS_K * 16]
    sums = tbl.reshape(N_WORKERS, 3, N_BINS_K, 16).sum(axis=(0, 3))
    count = sums[0]
    acc_sum = sums[1]
    conf_sum = sums[2]
    prop = count / jnp.float32(N_TOTAL)
    safe = jnp.maximum(count, 1.0)
    contrib = jnp.abs(conf_sum / safe - acc_sum / safe) * prop
    ece = jnp.sum(jnp.where(count > 0, contrib, 0.0))
    return ece.reshape(1)


# speedup vs baseline: 1.2393x; 1.2393x over previous
import functools

import jax
import jax.numpy as jnp
from jax import lax
from jax.experimental import pallas as pl
from jax.experimental.pallas import tpu as pltpu
from jax.experimental.pallas import tpu_sc as plsc

N_TOTAL = 4_000_000
N_BINS_K = 15
N_WORKERS = 32
PER_W = N_TOTAL // N_WORKERS
CHUNK_A = 12_504
CHUNK_B = PER_W - 9 * CHUNK_A
BUF = 12_544
VREGS = BUF // 16
UNROLL = 8
TBL = 256

_mesh = plsc.VectorSubcoreMesh(core_axis_name="c", subcore_axis_name="s")


@functools.partial(
    pl.kernel,
    out_type=(
        jax.ShapeDtypeStruct((N_WORKERS * TBL,), jnp.int32),
        jax.ShapeDtypeStruct((N_WORKERS * TBL,), jnp.float32),
    ),
    mesh=_mesh,
    compiler_params=pltpu.CompilerParams(needs_layout_passes=False),
    scratch_types=[
        pltpu.VMEM((BUF,), jnp.int32),
        pltpu.VMEM((BUF,), jnp.int32),
        pltpu.VMEM((BUF,), jnp.int32),
        pltpu.VMEM((BUF,), jnp.int32),
        pltpu.VMEM((BUF,), jnp.float32),
        pltpu.VMEM((BUF,), jnp.float32),
        pltpu.VMEM((TBL,), jnp.int32),
        pltpu.VMEM((TBL,), jnp.float32),
        pltpu.SemaphoreType.DMA,
        pltpu.SemaphoreType.DMA,
    ],
)
def _ece_hist(preds_hbm, targets_hbm, confs_hbm, out_i_hbm, out_f_hbm,
              p0, p1, t0, t1, c0, c1, tbl_i, tbl_f, sem0, sem1):
    wid = lax.axis_index("s") * 2 + lax.axis_index("c")
    base = wid * PER_W
    pbufs, tbufs, cbufs = (p0, p1), (t0, t1), (c0, c1)
    sems = (sem0, sem1)

    zero_i = jnp.zeros((16,), jnp.int32)
    zero_f = jnp.zeros((16,), jnp.float32)
    for k in range(TBL // 16):
        tbl_i[pl.ds(k * 16, 16)] = zero_i
        tbl_f[pl.ds(k * 16, 16)] = zero_f
    for off in (12_496, 12_512, 12_528):
        c0[pl.ds(off, 16)] = zero_f
        c1[pl.ds(off, 16)] = zero_f

    lane = lax.iota(jnp.int32, 16)
    c15 = jnp.float32(N_BINS_K)
    inv15 = jnp.float32(1.0 / N_BINS_K)

    def start(c, par):
        size = CHUNK_A if c < 9 else CHUNK_B
        off = base + c * CHUNK_A
        return (
            pltpu.async_copy(preds_hbm.at[pl.ds(off, size)],
                             pbufs[par].at[pl.ds(0, size)], sems[par]),
            pltpu.async_copy(targets_hbm.at[pl.ds(off, size)],
                             tbufs[par].at[pl.ds(0, size)], sems[par]),
            pltpu.async_copy(confs_hbm.at[pl.ds(off, size)],
                             cbufs[par].at[pl.ds(0, size)], sems[par]),
        )

    def compute(par, n_iters):
        p_v, t_v, c_v = pbufs[par], tbufs[par], cbufs[par]

        def vbody(j, carry):
            for u in range(UNROLL):
                b0 = (j * UNROLL + u) * 16
                conf = c_v[pl.ds(b0, 16)]
                p = p_v[pl.ds(b0, 16)]
                t = t_v[pl.ds(b0, 16)]
                ji = jnp.minimum((conf * c15).astype(jnp.int32), N_BINS_K - 1)
                jf = ji.astype(jnp.float32)
                blo = jf * inv15
                bhi = (jf + 1.0) * inv15
                binx = (ji
                        - jnp.where(conf <= blo, 1, 0)
                        + jnp.where(conf > bhi, 1, 0))
                valid = binx >= 0
                sidx = binx * 16 + lane
                vi = jnp.where(p == t, jnp.int32(65537), jnp.int32(1))
                plsc.addupdate_scatter(tbl_i, [sidx], vi, mask=valid)
                plsc.addupdate_scatter(tbl_f, [sidx], conf, mask=valid)
            return carry

        lax.fori_loop(0, n_iters, vbody, 0)

    handles = [None, None]
    handles[0] = start(0, 0)
    for c in range(10):
        par = c & 1
        if c + 1 < 10:
            handles[1 - par] = start(c + 1, 1 - par)
        for h in handles[par]:
            h.wait()
        if c == 9:
            for off in (12_464, 12_480, 12_496, 12_512, 12_528):
                cbufs[par][pl.ds(off, 16)] = zero_f
        compute(par, VREGS // UNROLL)

    pltpu.sync_copy(tbl_i, out_i_hbm.at[pl.ds(wid * TBL, TBL)])
    pltpu.sync_copy(tbl_f, out_f_hbm.at[pl.ds(wid * TBL, TBL)])


def kernel(preds, targets, confs):
    raw_i, raw_f = _ece_hist(preds.astype(jnp.int32),
                             targets.astype(jnp.int32), confs)
    pk = raw_i.reshape(N_WORKERS, TBL)[:, :N_BINS_K * 16]
    pk = pk.reshape(N_WORKERS, N_BINS_K, 16).sum(axis=(0, 2))
    count = (pk & 0xFFFF).astype(jnp.float32)
    acc_sum = (pk >> 16).astype(jnp.float32)
    cf = raw_f.reshape(N_WORKERS, TBL)[:, :N_BINS_K * 16]
    conf_sum = cf.reshape(N_WORKERS, N_BINS_K, 16).sum(axis=(0, 2))
    prop = count / jnp.float32(N_TOTAL)
    safe = jnp.maximum(count, 1.0)
    contrib = jnp.abs(conf_sum / safe - acc_sum / safe) * prop
    ece = jnp.sum(jnp.where(count > 0, contrib, 0.0))
    return ece.reshape(1)


# speedup vs baseline: 3.5330x; 2.8508x over previous
import functools

import jax
import jax.numpy as jnp
from jax import lax
from jax.experimental import pallas as pl
from jax.experimental.pallas import tpu as pltpu
from jax.experimental.pallas import tpu_sc as plsc

N_TOTAL = 4_000_000
N_BINS_K = 15
N_WORKERS = 32
PER_W = N_TOTAL // N_WORKERS
CHUNK_A = 12_504
CHUNK_B = PER_W - 9 * CHUNK_A
BUF = 12_544
VREGS = BUF // 16
UNROLL = 8
TBL = 256

_mesh = plsc.VectorSubcoreMesh(core_axis_name="c", subcore_axis_name="s")


@functools.partial(
    pl.kernel,
    out_type=(
        jax.ShapeDtypeStruct((N_WORKERS * TBL,), jnp.int32),
        jax.ShapeDtypeStruct((N_WORKERS * TBL,), jnp.float32),
    ),
    mesh=_mesh,
    compiler_params=pltpu.CompilerParams(needs_layout_passes=False),
    scratch_types=[
        pltpu.VMEM((BUF,), jnp.int32),
        pltpu.VMEM((BUF,), jnp.int32),
        pltpu.VMEM((BUF,), jnp.int32),
        pltpu.VMEM((BUF,), jnp.int32),
        pltpu.VMEM((BUF,), jnp.float32),
        pltpu.VMEM((BUF,), jnp.float32),
        pltpu.VMEM((TBL,), jnp.int32),
        pltpu.VMEM((TBL,), jnp.float32),
        pltpu.SemaphoreType.DMA,
        pltpu.SemaphoreType.DMA,
    ],
)
def _ece_hist(preds_hbm, targets_hbm, confs_hbm, out_i_hbm, out_f_hbm,
              p0, p1, t0, t1, c0, c1, tbl_i, tbl_f, sem0, sem1):
    wid = lax.axis_index("s") * 2 + lax.axis_index("c")
    base = wid * PER_W
    pbufs, tbufs, cbufs = (p0, p1), (t0, t1), (c0, c1)
    sems = (sem0, sem1)

    zero_i = jnp.zeros((16,), jnp.int32)
    zero_f = jnp.zeros((16,), jnp.float32)
    for k in range(TBL // 16):
        tbl_i[pl.ds(k * 16, 16)] = zero_i
        tbl_f[pl.ds(k * 16, 16)] = zero_f
    for off in (12_496, 12_512, 12_528):
        c0[pl.ds(off, 16)] = zero_f
        c1[pl.ds(off, 16)] = zero_f

    lane = lax.iota(jnp.int32, 16)
    c15 = jnp.float32(N_BINS_K)
    inv15 = jnp.float32(1.0 / N_BINS_K)

    def start(c, par):
        size = CHUNK_A if c < 9 else CHUNK_B
        off = base + c * CHUNK_A
        return (
            pltpu.async_copy(preds_hbm.at[pl.ds(off, size)],
                             pbufs[par].at[pl.ds(0, size)], sems[par]),
            pltpu.async_copy(targets_hbm.at[pl.ds(off, size)],
                             tbufs[par].at[pl.ds(0, size)], sems[par]),
            pltpu.async_copy(confs_hbm.at[pl.ds(off, size)],
                             cbufs[par].at[pl.ds(0, size)], sems[par]),
        )

    def compute(par, n_vregs):
        p_v, t_v, c_v = pbufs[par], tbufs[par], cbufs[par]

        @plsc.parallel_loop(0, n_vregs, 1, unroll=UNROLL)
        def vbody(j):
            b0 = j * 16
            conf = c_v[pl.ds(b0, 16)]
            p = p_v[pl.ds(b0, 16)]
            t = t_v[pl.ds(b0, 16)]
            ji = jnp.minimum((conf * c15).astype(jnp.int32), N_BINS_K - 1)
            jf = ji.astype(jnp.float32)
            blo = jf * inv15
            bhi = (jf + 1.0) * inv15
            binx = (ji
                    - jnp.where(conf <= blo, 1, 0)
                    + jnp.where(conf > bhi, 1, 0))
            valid = binx >= 0
            sidx = binx * 16 + lane
            vi = jnp.where(p == t, jnp.int32(65537), jnp.int32(1))
            plsc.addupdate_scatter(tbl_i, [sidx], vi, mask=valid)
            plsc.addupdate_scatter(tbl_f, [sidx], conf, mask=valid)

    handles = [None, None]
    handles[0] = start(0, 0)
    for c in range(10):
        par = c & 1
        if c + 1 < 10:
            handles[1 - par] = start(c + 1, 1 - par)
        for h in handles[par]:
            h.wait()
        if c == 9:
            for off in (12_464, 12_480, 12_496, 12_512, 12_528):
                cbufs[par][pl.ds(off, 16)] = zero_f
        compute(par, VREGS)

    pltpu.sync_copy(tbl_i, out_i_hbm.at[pl.ds(wid * TBL, TBL)])
    pltpu.sync_copy(tbl_f, out_f_hbm.at[pl.ds(wid * TBL, TBL)])


def kernel(preds, targets, confs):
    raw_i, raw_f = _ece_hist(preds.astype(jnp.int32),
                             targets.astype(jnp.int32), confs)
    pk = raw_i.reshape(N_WORKERS, TBL)[:, :N_BINS_K * 16]
    pk = pk.reshape(N_WORKERS, N_BINS_K, 16).sum(axis=(0, 2))
    count = (pk & 0xFFFF).astype(jnp.float32)
    acc_sum = (pk >> 16).astype(jnp.float32)
    cf = raw_f.reshape(N_WORKERS, TBL)[:, :N_BINS_K * 16]
    conf_sum = cf.reshape(N_WORKERS, N_BINS_K, 16).sum(axis=(0, 2))
    prop = count / jnp.float32(N_TOTAL)
    safe = jnp.maximum(count, 1.0)
    contrib = jnp.abs(conf_sum / safe - acc_sum / safe) * prop
    ece = jnp.sum(jnp.where(count > 0, contrib, 0.0))
    return ece.reshape(1)
